# baseline (speedup 1.0000x reference)
import functools

import jax
import jax.numpy as jnp
from jax import lax
from jax.experimental import pallas as pl
from jax.experimental.pallas import tpu as pltpu
from jax.experimental.pallas import tpu_sc as plsc

B, D, N = 1024, 128, 16384
MULT = N // B
ROW = D * MULT
L = 16
NC, NS = 2, 16
NW = NC * NS
RPW = B // NW
ND = D // L
NV_ROW = ROW // L
NV_N = N // L
ZERO_POINT = 1e-10
FN = float(N)
FM = float(MULT)


def _body(zm_hbm, zv_hbm, x_hbm, eps_hbm, out_hbm,
          hist, zrow, idxrow, zm_all, zv_all,
          eps_v0, x_v0, o_v0,
          eps_v1, x_v1, o_v1,
          psem, isem0, isem1, osem0, osem1):
    cid = lax.axis_index("c")
    sid = lax.axis_index("s")
    wid = sid * NC + cid
    row0 = wid * RPW

    bufs = ((eps_v0, x_v0, o_v0, isem0, osem0),
            (eps_v1, x_v1, o_v1, isem1, osem1))

    zeros16 = jnp.zeros((L,), jnp.float32)

    pltpu.async_copy(zm_hbm.at[pl.ds(row0, RPW)], zm_all, psem)
    pltpu.async_copy(zv_hbm.at[pl.ds(row0, RPW)], zv_all, psem)

    @plsc.parallel_loop(0, NV_N, unroll=8)
    def _zero(i):
        hist[pl.ds(i * L, L)] = zeros16

    def _start_in(b, bb):
        eps_v, x_v, _, isem, _ = bufs[bb]
        pltpu.async_copy(eps_hbm.at[pl.ds(b * MULT, MULT)], eps_v, isem)
        pltpu.async_copy(x_hbm.at[b], x_v, isem)

    _start_in(row0, 0)
    _start_in(row0 + 1, 1)

    pltpu.make_async_copy(zm_hbm.at[pl.ds(row0, RPW)], zm_all, psem).wait()
    pltpu.make_async_copy(zv_hbm.at[pl.ds(row0, RPW)], zv_all, psem).wait()

    def _pair(g, carry):
        for bb in (0, 1):
            eps_v, x_v, o_v, isem, osem = bufs[bb]
            r = 2 * g + bb
            b = row0 + r

            pltpu.make_async_copy(eps_hbm.at[pl.ds(b * MULT, MULT)],
                                  eps_v, isem).wait()

            zmv = [zm_all[r, pl.ds(d * L, L)] for d in range(ND)]
            emv = [jnp.exp(0.5 * zv_all[r, pl.ds(d * L, L)])
                   for d in range(ND)]

            inf16 = jnp.full((L,), jnp.inf, jnp.float32)

            def _zk(k, mm):
                mn0, mn1, mx0, mx1 = mm
                for d in range(ND):
                    z = zmv[d] + emv[d] * eps_v[k, pl.ds(d * L, L)]
                    zrow[pl.ds(k * D + d * L, L)] = z
                    if d % 2 == 0:
                        mn0 = jnp.minimum(mn0, z)
                        mx0 = jnp.maximum(mx0, z)
                    else:
                        mn1 = jnp.minimum(mn1, z)
                        mx1 = jnp.maximum(mx1, z)
                return mn0, mn1, mx0, mx1

            mn0, mn1, mx0, mx1 = plsc.parallel_loop(
                0, MULT, unroll=4,
                carry=(inf16, inf16, -inf16, -inf16))(_zk)
            vmin = -jnp.max(-jnp.minimum(mn0, mn1))
            vmax = jnp.max(jnp.maximum(mx0, mx1))
            vminv = jnp.broadcast_to(vmin, (L,))
            vmaxv = jnp.broadcast_to(vmax, (L,))
            widthv = (vmaxv - vminv) / FN

            ones16 = zeros16 + 1.0

            @plsc.parallel_loop(0, NV_ROW, unroll=8)
            def _hk(j):
                off = j * L
                z = zrow[pl.ds(off, L)]
                ii = ((z - vminv) / widthv).astype(jnp.int32)
                ii = jnp.clip(ii, 0, N - 1)
                idxrow[pl.ds(off, L)] = ii
                plsc.addupdate_scatter(hist, [ii], ones16)

            def _g1(j, mm):
                m0, m1 = mm
                off = j * (2 * L)
                c0 = plsc.load_gather(hist, [idxrow[pl.ds(off, L)]])
                c1 = plsc.load_gather(hist, [idxrow[pl.ds(off + L, L)]])
                zrow[pl.ds(off, L)] = c0
                zrow[pl.ds(off + L, L)] = c1
                return jnp.maximum(m0, c0), jnp.maximum(m1, c1)
            m0, m1 = plsc.parallel_loop(
                0, NV_ROW // 2, unroll=4, carry=(zeros16, zeros16))(_g1)
            cmaxv = jnp.broadcast_to(jnp.max(jnp.maximum(m0, m1)), (L,))

            def _g2(j, acc):
                a10, a11, a20, a21 = acc
                off = j * (2 * L)
                cc0 = zrow[pl.ds(off, L)]
                cc1 = zrow[pl.ds(off + L, L)]
                e0 = jnp.exp(cc0 - cmaxv)
                e1 = jnp.exp(cc1 - cmaxv)
                zrow[pl.ds(off, L)] = e0
                zrow[pl.ds(off + L, L)] = e1
                r0 = 1.0 / cc0
                r1 = 1.0 / cc1
                return a10 + e0 * r0, a11 + e1 * r1, a20 + r0, a21 + r1
            a10, a11, a20, a21 = plsc.parallel_loop(
                0, NV_ROW // 2, unroll=4, carry=(zeros16,) * 4)(_g2)
            e0v = jnp.exp(zeros16 - cmaxv)
            denomv = (jnp.broadcast_to(jnp.sum(a10 + a11), (L,))
                      + (FN - jnp.broadcast_to(jnp.sum(a20 + a21), (L,)))
                      * e0v)
            invdenomv = 1.0 / denomv
            p0v = e0v * invdenomv
            sv = jnp.where(p0v < ZERO_POINT, 0.0, FM * p0v)

            pltpu.make_async_copy(x_hbm.at[b], x_v, isem).wait()

            @pl.when(g > 0)
            def _wait_out():
                pltpu.make_async_copy(o_v, out_hbm.at[b], osem).wait()

            @plsc.parallel_loop(0, NV_N, unroll=8)
            def _dk(j):
                off = j * L
                o_v[pl.ds(off, L)] = x_v[pl.ds(off, L)] * sv

            @plsc.parallel_loop(0, NV_ROW, unroll=8)
            def _ck(j):
                off = j * L
                ii = idxrow[pl.ds(off, L)]
                e = zrow[pl.ds(off, L)]
                p = e * invdenomv
                p = jnp.where(p < ZERO_POINT, 0.0, p)
                xg = plsc.load_gather(x_v, [ii])
                plsc.store_scatter(o_v, [ii], xg * (FM * p))
                plsc.store_scatter(hist, [ii], zeros16)

            pltpu.async_copy(o_v, out_hbm.at[b], osem)

            @pl.when(g < RPW // 2 - 1)
            def _prefetch():
                _start_in(b + 2, bb)
        return carry

    lax.fori_loop(0, RPW // 2, _pair, 0)

    pltpu.make_async_copy(o_v0, out_hbm.at[row0 + RPW - 2], osem0).wait()
    pltpu.make_async_copy(o_v1, out_hbm.at[row0 + RPW - 1], osem1).wait()


@jax.jit
def _run(z_mean, z_var, x, epsilon):
    mesh = plsc.VectorSubcoreMesh(core_axis_name="c", subcore_axis_name="s")
    dbl = [
        pltpu.VMEM((MULT, D), jnp.float32),
        pltpu.VMEM((N,), jnp.float32),
        pltpu.VMEM((N,), jnp.float32),
    ]
    f = functools.partial(
        pl.kernel,
        out_type=jax.ShapeDtypeStruct((B, N), jnp.float32),
        mesh=mesh,
        scratch_types=[
            pltpu.VMEM((N,), jnp.float32),
            pltpu.VMEM((ROW,), jnp.float32),
            pltpu.VMEM((ROW,), jnp.int32),
            pltpu.VMEM((RPW, D), jnp.float32),
            pltpu.VMEM((RPW, D), jnp.float32),
        ] + dbl + dbl + [
            pltpu.SemaphoreType.DMA,
            pltpu.SemaphoreType.DMA,
            pltpu.SemaphoreType.DMA,
            pltpu.SemaphoreType.DMA,
            pltpu.SemaphoreType.DMA,
        ],
        compiler_params=pltpu.CompilerParams(needs_layout_passes=False),
    )(_body)
    return f(z_mean, z_var, x, epsilon)


def kernel(z_mean, z_var, x, epsilon):
    return _run(z_mean, z_var, x, epsilon)


# speedup vs baseline: 1.0049x; 1.0049x over previous
import functools

import jax
import jax.numpy as jnp
from jax import lax
from jax.experimental import pallas as pl
from jax.experimental.pallas import tpu as pltpu
from jax.experimental.pallas import tpu_sc as plsc

B, D, N = 1024, 128, 16384
MULT = N // B
ROW = D * MULT
L = 16
NC, NS = 2, 16
NW = NC * NS
RPW = B // NW
ND = D // L
NV_ROW = ROW // L
NV_N = N // L
ZERO_POINT = 1e-10
FN = float(N)
FM = float(MULT)


def _body(zm_hbm, zv_hbm, x_hbm, eps_hbm, out_hbm,
          hist, zrow, idxrow, zm_all, zv_all,
          eps_v0, x_v0, o_v0,
          eps_v1, x_v1, o_v1,
          psem, isem0, isem1, osem0, osem1):
    cid = lax.axis_index("c")
    sid = lax.axis_index("s")
    wid = sid * NC + cid
    row0 = wid * RPW

    bufs = ((eps_v0, x_v0, o_v0, isem0, osem0),
            (eps_v1, x_v1, o_v1, isem1, osem1))

    zeros16 = jnp.zeros((L,), jnp.float32)

    pltpu.async_copy(zm_hbm.at[pl.ds(row0, RPW)], zm_all, psem)
    pltpu.async_copy(zv_hbm.at[pl.ds(row0, RPW)], zv_all, psem)

    @plsc.parallel_loop(0, NV_N, unroll=8)
    def _zero(i):
        hist[pl.ds(i * L, L)] = zeros16

    def _start_in(b, bb):
        eps_v, x_v, _, isem, _ = bufs[bb]
        pltpu.async_copy(eps_hbm.at[pl.ds(b * MULT, MULT)], eps_v, isem)
        pltpu.async_copy(x_hbm.at[b], x_v, isem)

    _start_in(row0, 0)
    _start_in(row0 + 1, 1)

    pltpu.make_async_copy(zm_hbm.at[pl.ds(row0, RPW)], zm_all, psem).wait()
    pltpu.make_async_copy(zv_hbm.at[pl.ds(row0, RPW)], zv_all, psem).wait()

    def _pair(g, carry):
        for bb in (0, 1):
            eps_v, x_v, o_v, isem, osem = bufs[bb]
            r = 2 * g + bb
            b = row0 + r

            pltpu.make_async_copy(eps_hbm.at[pl.ds(b * MULT, MULT)],
                                  eps_v, isem).wait()

            zmv = [zm_all[r, pl.ds(d * L, L)] for d in range(ND)]
            emv = [jnp.exp(0.5 * zv_all[r, pl.ds(d * L, L)])
                   for d in range(ND)]

            inf16 = jnp.full((L,), jnp.inf, jnp.float32)

            def _zk(k, mm):
                mn0, mn1, mx0, mx1 = mm
                for d in range(ND):
                    z = zmv[d] + emv[d] * eps_v[k, pl.ds(d * L, L)]
                    zrow[pl.ds(k * D + d * L, L)] = z
                    if d % 2 == 0:
                        mn0 = jnp.minimum(mn0, z)
                        mx0 = jnp.maximum(mx0, z)
                    else:
                        mn1 = jnp.minimum(mn1, z)
                        mx1 = jnp.maximum(mx1, z)
                return mn0, mn1, mx0, mx1

            mn0, mn1, mx0, mx1 = plsc.parallel_loop(
                0, MULT, unroll=4,
                carry=(inf16, inf16, -inf16, -inf16))(_zk)
            vmin = -jnp.max(-jnp.minimum(mn0, mn1))
            vmax = jnp.max(jnp.maximum(mx0, mx1))
            vminv = jnp.broadcast_to(vmin, (L,))
            vmaxv = jnp.broadcast_to(vmax, (L,))
            widthv = (vmaxv - vminv) / FN

            ones16 = zeros16 + 1.0

            @plsc.parallel_loop(0, NV_ROW, unroll=8)
            def _hk(j):
                off = j * L
                z = zrow[pl.ds(off, L)]
                ii = ((z - vminv) / widthv).astype(jnp.int32)
                ii = jnp.clip(ii, 0, N - 1)
                idxrow[pl.ds(off, L)] = ii
                plsc.addupdate_scatter(hist, [ii], ones16)

            def _g1(j, mm):
                m0, m1 = mm
                off = j * (2 * L)
                c0 = plsc.load_gather(hist, [idxrow[pl.ds(off, L)]])
                c1 = plsc.load_gather(hist, [idxrow[pl.ds(off + L, L)]])
                zrow[pl.ds(off, L)] = c0
                zrow[pl.ds(off + L, L)] = c1
                return jnp.maximum(m0, c0), jnp.maximum(m1, c1)
            m0, m1 = plsc.parallel_loop(
                0, NV_ROW // 2, unroll=4, carry=(zeros16, zeros16))(_g1)
            cmaxv = jnp.broadcast_to(jnp.max(jnp.maximum(m0, m1)), (L,))

            def _g2(j, acc):
                a10, a11, a20, a21 = acc
                off = j * (2 * L)
                cc0 = zrow[pl.ds(off, L)]
                cc1 = zrow[pl.ds(off + L, L)]
                e0 = jnp.exp(cc0 - cmaxv)
                e1 = jnp.exp(cc1 - cmaxv)
                zrow[pl.ds(off, L)] = e0
                zrow[pl.ds(off + L, L)] = e1
                r0 = 1.0 / cc0
                r1 = 1.0 / cc1
                return a10 + e0 * r0, a11 + e1 * r1, a20 + r0, a21 + r1
            a10, a11, a20, a21 = plsc.parallel_loop(
                0, NV_ROW // 2, unroll=4, carry=(zeros16,) * 4)(_g2)
            e0v = jnp.exp(zeros16 - cmaxv)
            denomv = (jnp.broadcast_to(jnp.sum(a10 + a11), (L,))
                      + (FN - jnp.broadcast_to(jnp.sum(a20 + a21), (L,)))
                      * e0v)
            invdenomv = 1.0 / denomv
            p0v = e0v * invdenomv
            sv = jnp.where(p0v < ZERO_POINT, 0.0, FM * p0v)

            pltpu.make_async_copy(x_hbm.at[b], x_v, isem).wait()

            @pl.when(g > 0)
            def _wait_out():
                pltpu.make_async_copy(o_v, out_hbm.at[b], osem).wait()

            @plsc.parallel_loop(0, NV_N, unroll=8)
            def _dk(j):
                off = j * L
                o_v[pl.ds(off, L)] = x_v[pl.ds(off, L)] * sv

            @plsc.parallel_loop(0, NV_ROW, unroll=4)
            def _ck(j):
                off = j * L
                ii = idxrow[pl.ds(off, L)]
                e = zrow[pl.ds(off, L)]
                p = e * invdenomv
                p = jnp.where(p < ZERO_POINT, 0.0, p)
                xg = plsc.load_gather(x_v, [ii])
                plsc.store_scatter(o_v, [ii], xg * (FM * p))
                plsc.store_scatter(hist, [ii], zeros16)

            pltpu.async_copy(o_v, out_hbm.at[b], osem)

            @pl.when(g < RPW // 2 - 1)
            def _prefetch():
                _start_in(b + 2, bb)
        return carry

    lax.fori_loop(0, RPW // 2, _pair, 0)

    pltpu.make_async_copy(o_v0, out_hbm.at[row0 + RPW - 2], osem0).wait()
    pltpu.make_async_copy(o_v1, out_hbm.at[row0 + RPW - 1], osem1).wait()


@jax.jit
def _run(z_mean, z_var, x, epsilon):
    mesh = plsc.VectorSubcoreMesh(core_axis_name="c", subcore_axis_name="s")
    dbl = [
        pltpu.VMEM((MULT, D), jnp.float32),
        pltpu.VMEM((N,), jnp.float32),
        pltpu.VMEM((N,), jnp.float32),
    ]
    f = functools.partial(
        pl.kernel,
        out_type=jax.ShapeDtypeStruct((B, N), jnp.float32),
        mesh=mesh,
        scratch_types=[
            pltpu.VMEM((N,), jnp.float32),
            pltpu.VMEM((ROW,), jnp.float32),
            pltpu.VMEM((ROW,), jnp.int32),
            pltpu.VMEM((RPW, D), jnp.float32),
            pltpu.VMEM((RPW, D), jnp.float32),
        ] + dbl + dbl + [
            pltpu.SemaphoreType.DMA,
            pltpu.SemaphoreType.DMA,
            pltpu.SemaphoreType.DMA,
            pltpu.SemaphoreType.DMA,
            pltpu.SemaphoreType.DMA,
        ],
        compiler_params=pltpu.CompilerParams(needs_layout_passes=False),
    )(_body)
    return f(z_mean, z_var, x, epsilon)


def kernel(z_mean, z_var, x, epsilon):
    return _run(z_mean, z_var, x, epsilon)
